# Initial kernel scaffold; baseline (speedup 1.0000x reference)
import jax
import jax.numpy as jnp
from jax.experimental import pallas as pl


def kernel(piece_indices, side_idx, castling_flags, en_passant_idx, halfmove, fullmove, repetitions, side_embed, castling_embed_K, castling_embed_Q, castling_embed_k, castling_embed_q, no_castling_embed, piece_embed, no_en_passant_embed, half_move_embed, full_move_embed, repetition_embed, pos_embed):
    raise NotImplementedError("write your pallas kernel here")


# speedup vs baseline: 1.1035x; 1.1035x over previous
import functools

import jax
import jax.numpy as jnp
from jax import lax
from jax.experimental import pallas as pl
from jax.experimental.pallas import tpu as pltpu
from jax.experimental.pallas import tpu_sc as plsc

B = 4096
H = 128
NT = 73
ROWS = B * NT
NW = 32
RPT = ROWS // NW
CHUNK = 128
NCH = RPT // CHUNK

OFF_FUSED = 0
OFF_SIDE = 832
OFF_CASTLE = 834
OFF_NOC = 838
OFF_POS = 839
OFF_NOEP = 903
OFF_HALF = 904
OFF_FULL = 1004
OFF_REP = 1260
TABLE_ROWS = 1264


def _prep_body(piece_ref, side_ref, castle_ref, ep_ref, hm_ref, fm_ref, rep_ref,
               piece_e_ref, pos_e_ref, fused_ref, idx_ref):
    pos = pos_e_ref[...]
    for p in range(13):
        fused_ref[p * 64:(p + 1) * 64, :] = pos + piece_e_ref[p:p + 1, :]

    board = piece_ref[...] * 64 + lax.broadcasted_iota(jnp.int32, (B, 64), 1)
    side_t = side_ref[...] + OFF_SIDE
    cast_t = jnp.where(castle_ref[...] > 0,
                       lax.broadcasted_iota(jnp.int32, (B, 4), 1) + OFF_CASTLE,
                       OFF_NOC)
    ep = ep_ref[...]
    ep_t = jnp.where(ep < 64, ep + OFF_POS, OFF_NOEP)
    hm_t = jnp.clip(hm_ref[...], 0, 99) + OFF_HALF
    fm_t = jnp.clip(fm_ref[...], 1, 256) - 1 + OFF_FULL
    rep_t = jnp.clip(rep_ref[...] - 1, 0, 2) + OFF_REP
    idx_ref[...] = jnp.concatenate(
        [board, side_t, cast_t, ep_t, hm_t, fm_t, rep_t], axis=1)


_prep = pl.pallas_call(
    _prep_body,
    out_shape=(jax.ShapeDtypeStruct((13 * 64, H), jnp.float32),
               jax.ShapeDtypeStruct((B, NT), jnp.int32)),
)


def _sc_gather_body(table_hbm, idx_hbm, out_hbm, idx_v, rows_v, sem):
    wid = lax.axis_index("s") * 2 + lax.axis_index("c")
    base0 = wid * RPT

    def body(i, carry):
        base = base0 + i * CHUNK
        pltpu.sync_copy(idx_hbm.at[pl.ds(base, CHUNK)], idx_v)
        pltpu.async_copy(table_hbm.at[idx_v], rows_v, sem).wait()
        pltpu.sync_copy(rows_v, out_hbm.at[pl.ds(base, CHUNK)])
        return carry

    lax.fori_loop(0, NCH, body, 0)


_sc_gather = pl.kernel(
    _sc_gather_body,
    out_type=jax.ShapeDtypeStruct((ROWS, H), jnp.float32),
    mesh=plsc.VectorSubcoreMesh(core_axis_name="c", subcore_axis_name="s"),
    scratch_types=[
        pltpu.VMEM((CHUNK,), jnp.int32),
        pltpu.VMEM((CHUNK, H), jnp.float32),
        pltpu.SemaphoreType.DMA,
    ],
)


def kernel(piece_indices, side_idx, castling_flags, en_passant_idx, halfmove,
           fullmove, repetitions, side_embed, castling_embed_K, castling_embed_Q,
           castling_embed_k, castling_embed_q, no_castling_embed, piece_embed,
           no_en_passant_embed, half_move_embed, full_move_embed,
           repetition_embed, pos_embed):
    i32 = jnp.int32
    fused, idx = _prep(
        piece_indices.astype(i32),
        side_idx.astype(i32).reshape(B, 1),
        castling_flags.astype(i32),
        en_passant_idx.astype(i32).reshape(B, 1),
        halfmove.astype(i32).reshape(B, 1),
        fullmove.astype(i32).reshape(B, 1),
        repetitions.astype(i32).reshape(B, 1),
        piece_embed, pos_embed)

    table = jnp.concatenate([
        fused,
        side_embed,
        castling_embed_K.reshape(1, H),
        castling_embed_Q.reshape(1, H),
        castling_embed_k.reshape(1, H),
        castling_embed_q.reshape(1, H),
        no_castling_embed.reshape(1, H),
        pos_embed,
        no_en_passant_embed.reshape(1, H),
        half_move_embed,
        full_move_embed,
        repetition_embed,
        jnp.zeros((1, H), jnp.float32),
    ], axis=0)

    flat = _sc_gather(table, idx.reshape(ROWS))
    return flat.reshape(B, NT, H)


# speedup vs baseline: 1.1909x; 1.0793x over previous
import functools

import jax
import jax.numpy as jnp
from jax import lax
from jax.experimental import pallas as pl
from jax.experimental.pallas import tpu as pltpu
from jax.experimental.pallas import tpu_sc as plsc

B = 4096
H = 128
NT = 73
ROWS = B * NT
NW = 32
RPT = ROWS // NW
CHUNK = 128
NCH = RPT // CHUNK

OFF_FUSED = 0
OFF_SIDE = 832
OFF_CASTLE = 834
OFF_NOC = 838
OFF_POS = 839
OFF_NOEP = 903
OFF_HALF = 904
OFF_FULL = 1004
OFF_REP = 1260
TABLE_ROWS = 1264


def _prep_body(piece_ref, side_ref, castle_ref, ep_ref, hm_ref, fm_ref, rep_ref,
               piece_e_ref, pos_e_ref, fused_ref, idx_ref):
    pos = pos_e_ref[...]
    for p in range(13):
        fused_ref[p * 64:(p + 1) * 64, :] = pos + piece_e_ref[p:p + 1, :]

    board = piece_ref[...] * 64 + lax.broadcasted_iota(jnp.int32, (B, 64), 1)
    side_t = side_ref[...] + OFF_SIDE
    cast_t = jnp.where(castle_ref[...] > 0,
                       lax.broadcasted_iota(jnp.int32, (B, 4), 1) + OFF_CASTLE,
                       OFF_NOC)
    ep = ep_ref[...]
    ep_t = jnp.where(ep < 64, ep + OFF_POS, OFF_NOEP)
    hm_t = jnp.clip(hm_ref[...], 0, 99) + OFF_HALF
    fm_t = jnp.clip(fm_ref[...], 1, 256) - 1 + OFF_FULL
    rep_t = jnp.clip(rep_ref[...] - 1, 0, 2) + OFF_REP
    idx_ref[...] = jnp.concatenate(
        [board, side_t, cast_t, ep_t, hm_t, fm_t, rep_t], axis=1)


_prep = pl.pallas_call(
    _prep_body,
    out_shape=(jax.ShapeDtypeStruct((13 * 64, H), jnp.float32),
               jax.ShapeDtypeStruct((B, NT), jnp.int32)),
)


NBUF = 4
NGRP = NCH // NBUF
TAIL = NCH - NGRP * NBUF


def _sc_gather_body(table_hbm, idx_hbm, out_hbm, idx_all,
                    rows0, rows1, rows2, rows3,
                    sg0, sg1, sg2, sg3, ss0, ss1, ss2, ss3):
    rows = (rows0, rows1, rows2, rows3)
    sg = (sg0, sg1, sg2, sg3)
    ss = (ss0, ss1, ss2, ss3)
    wid = lax.axis_index("s") * 2 + lax.axis_index("c")
    base0 = wid * RPT

    pltpu.sync_copy(idx_hbm.at[wid], idx_all)

    def group(g, carry):
        c0 = g * NBUF
        gathers = [pltpu.async_copy(table_hbm.at[idx_all.at[c0 + b]],
                                    rows[b], sg[b])
                   for b in range(NBUF)]
        stores = []
        for b in range(NBUF):
            gathers[b].wait()
            stores.append(pltpu.async_copy(
                rows[b], out_hbm.at[pl.ds(base0 + (c0 + b) * CHUNK, CHUNK)],
                ss[b]))
        for b in range(NBUF):
            stores[b].wait()
        return carry

    lax.fori_loop(0, NGRP, group, 0)

    for t in range(TAIL):
        c = NGRP * NBUF + t
        pltpu.async_copy(table_hbm.at[idx_all.at[c]], rows[t], sg[t]).wait()
        pltpu.sync_copy(rows[t], out_hbm.at[pl.ds(base0 + c * CHUNK, CHUNK)])


_sc_gather = pl.kernel(
    _sc_gather_body,
    out_type=jax.ShapeDtypeStruct((ROWS, H), jnp.float32),
    mesh=plsc.VectorSubcoreMesh(core_axis_name="c", subcore_axis_name="s"),
    scratch_types=(
        [pltpu.VMEM((NCH, CHUNK), jnp.int32)]
        + [pltpu.VMEM((CHUNK, H), jnp.float32) for _ in range(NBUF)]
        + [pltpu.SemaphoreType.DMA for _ in range(2 * NBUF)]
    ),
)


def kernel(piece_indices, side_idx, castling_flags, en_passant_idx, halfmove,
           fullmove, repetitions, side_embed, castling_embed_K, castling_embed_Q,
           castling_embed_k, castling_embed_q, no_castling_embed, piece_embed,
           no_en_passant_embed, half_move_embed, full_move_embed,
           repetition_embed, pos_embed):
    i32 = jnp.int32
    fused, idx = _prep(
        piece_indices.astype(i32),
        side_idx.astype(i32).reshape(B, 1),
        castling_flags.astype(i32),
        en_passant_idx.astype(i32).reshape(B, 1),
        halfmove.astype(i32).reshape(B, 1),
        fullmove.astype(i32).reshape(B, 1),
        repetitions.astype(i32).reshape(B, 1),
        piece_embed, pos_embed)

    table = jnp.concatenate([
        fused,
        side_embed,
        castling_embed_K.reshape(1, H),
        castling_embed_Q.reshape(1, H),
        castling_embed_k.reshape(1, H),
        castling_embed_q.reshape(1, H),
        no_castling_embed.reshape(1, H),
        pos_embed,
        no_en_passant_embed.reshape(1, H),
        half_move_embed,
        full_move_embed,
        repetition_embed,
        jnp.zeros((1, H), jnp.float32),
    ], axis=0)

    flat = _sc_gather(table, idx.reshape(NW, NCH, CHUNK))
    return flat.reshape(B, NT, H)
